# Initial kernel scaffold; baseline (speedup 1.0000x reference)
import jax
import jax.numpy as jnp
from jax.experimental import pallas as pl


def kernel(h, x, rel_x, r_feat, edge_feat, invar_ligand_shape, ligand_shape_emb, topo_out, e_w, params, edge_index):
    raise NotImplementedError("write your pallas kernel here")


# speedup vs baseline: 24.8939x; 24.8939x over previous
import functools
import math

import jax
import jax.numpy as jnp
from jax import lax
from jax.experimental import pallas as pl
from jax.experimental.pallas import tpu as pltpu
from jax.experimental.pallas import tpu_sc as plsc

_NW = 32
_W = 80
_LN_EPS = 1e-5
_VN_EPS = 1e-6


def _node_body(h_ref, topo_ref, invar_ref,
               wkh_ref, wkt_ref, wki_ref, b1k_ref,
               wvh_ref, wvt_ref, wvi_ref, b1v_ref,
               w1q_ref, b1q_ref, gq_ref, betaq_ref, w2q_ref, b2q_ref,
               out_ref):
    h = h_ref[...]
    topo = topo_ref[...]
    invar = invar_ref[...]
    adk = h @ wkh_ref[...] + topo @ wkt_ref[...] + invar @ wki_ref[...] + b1k_ref[...]
    adv = h @ wvh_ref[...] + topo @ wvt_ref[...] + invar @ wvi_ref[...] + b1v_ref[...]
    y = h @ w1q_ref[...] + b1q_ref[...]
    mu = jnp.mean(y, axis=-1, keepdims=True)
    var = jnp.mean((y - mu) ** 2, axis=-1, keepdims=True)
    z = (y - mu) * lax.rsqrt(var + _LN_EPS) * gq_ref[...] + betaq_ref[...]
    z = jnp.maximum(z, 0.0)
    q = z @ w2q_ref[...] + b2q_ref[...]
    out_ref[...] = jnp.concatenate([adk, adv, q], axis=1)


def _ln_relu(y, g, beta):
    mu = jnp.mean(y, axis=-1, keepdims=True)
    var = jnp.mean((y - mu) ** 2, axis=-1, keepdims=True)
    z = (y - mu) * lax.rsqrt(var + _LN_EPS) * g + beta
    return jnp.maximum(z, 0.0)


def _edge_body(gd_ref, gh_ref, ef_ref, rf_ref, ew_ref, rx_ref,
               wsrc_ref, wef_ref,
               gk_ref, betak_ref, w2k_ref, b2k_ref,
               gv_ref, betav_ref, w2v_ref, b2v_ref, s_ref,
               out_ref):
    gd = gd_ref[...]
    sp = gh_ref[...] @ wsrc_ref[...]
    efrf = jnp.concatenate([ef_ref[...], rf_ref[...]], axis=1)
    ep = efrf @ wef_ref[...]
    y1k = gd[:, 0:128] + sp[:, 0:128] + ep[:, 0:128]
    y1v = gd[:, 128:256] + sp[:, 128:256] + ep[:, 128:256]
    zk = _ln_relu(y1k, gk_ref[...], betak_ref[...])
    zv = _ln_relu(y1v, gv_ref[...], betav_ref[...])
    kk = zk @ w2k_ref[...] + b2k_ref[...]
    vs = (zv @ w2v_ref[...] + b2v_ref[...]) * ew_ref[...]
    qg = gd[:, 256:384]
    ex = jnp.exp((qg * kk) @ s_ref[...])
    w = ex * vs
    rx = rx_ref[...]
    pad = jnp.zeros((ex.shape[0], 64), jnp.float32)
    out_ref[...] = jnp.concatenate(
        [ex, w * rx[:, 0:1], w * rx[:, 1:2], w * rx[:, 2:3], pad], axis=1)


def _final_body(acc_ref, x_ref, lse_ref,
                wf0_ref, wfo_ref, wfl_ref, wd0_ref, wdo_ref, wdl_ref,
                out_ref):
    a = acc_ref[0] + acc_ref[1]
    denom = a[:, 0:16] + 1e-16
    ods, ps, ds_ = [], [], []
    for d in range(3):
        od = a[:, 16 * (d + 1):16 * (d + 2)] / denom
        xd = x_ref[:, d:d + 1]
        ld = lse_ref[d]
        ods.append(od)
        ps.append(xd * wf0_ref[...] + od @ wfo_ref[...] + ld @ wfl_ref[...])
        ds_.append(xd * wd0_ref[...] + od @ wdo_ref[...] + ld @ wdl_ref[...])
    dot = ps[0] * ds_[0] + ps[1] * ds_[1] + ps[2] * ds_[2]
    dns = ds_[0] * ds_[0] + ds_[1] * ds_[1] + ds_[2] * ds_[2]
    coef = dot / (dns + _VN_EPS)
    fneg = jnp.where(dot >= 0, 0.0, 0.8)
    cols = []
    for d in range(3):
        vn = ps[d] - fneg * coef * ds_[d]
        cols.append(jnp.mean(ods[d], axis=1, keepdims=True)
                    + jnp.mean(vn, axis=1, keepdims=True))
    out_ref[...] = jnp.concatenate(cols, axis=1)


def _sc_gather(tdst, hsrc, dst, src):
    e = dst.shape[0]
    epw = e // _NW
    nwin = epw // _W
    dt = tdst.shape[1]
    dh = hsrc.shape[1]
    mesh = plsc.VectorSubcoreMesh(core_axis_name="c", subcore_axis_name="s")

    @functools.partial(
        pl.kernel,
        out_type=[jax.ShapeDtypeStruct((e, dt), jnp.float32),
                  jax.ShapeDtypeStruct((e, dh), jnp.float32)],
        mesh=mesh,
        scratch_types=[
            pltpu.VMEM((_W,), jnp.int32),
            pltpu.VMEM((_W,), jnp.int32),
            pltpu.VMEM((_W, dt), jnp.float32),
            pltpu.VMEM((_W, dh), jnp.float32),
            pltpu.SemaphoreType.DMA,
            pltpu.SemaphoreType.DMA,
        ])
    def run(tdst_hbm, h_hbm, di_hbm, si_hbm, gd_hbm, gh_hbm,
            dib, sib, drows, hrows, sem_d, sem_s):
        cid = lax.axis_index("c")
        sid = lax.axis_index("s")
        wid = sid * 2 + cid
        base0 = wid * epw

        def win(w, carry):
            base = base0 + w * _W
            pltpu.sync_copy(di_hbm.at[pl.ds(base, _W)], dib)
            pltpu.sync_copy(si_hbm.at[pl.ds(base, _W)], sib)
            cp1 = pltpu.async_copy(tdst_hbm.at[dib], drows, sem_d)
            cp2 = pltpu.async_copy(h_hbm.at[sib], hrows, sem_s)
            cp1.wait()
            cp2.wait()
            pltpu.sync_copy(drows, gd_hbm.at[pl.ds(base, _W)])
            pltpu.sync_copy(hrows, gh_hbm.at[pl.ds(base, _W)])
            return carry

        lax.fori_loop(0, nwin, win, 0)

    return run(tdst, hsrc, dst, src)


def _sc_scatter(m, dst, zeros, n):
    e = dst.shape[0]
    epw = e // _NW
    nwin = epw // _W
    npt = zeros.shape[0]
    n_pad = npt * 16
    mesh = plsc.VectorSubcoreMesh(core_axis_name="c", subcore_axis_name="s")

    @functools.partial(
        pl.kernel,
        out_type=jax.ShapeDtypeStruct((2, n_pad, 128), jnp.float32),
        mesh=mesh,
        scratch_types=[
            pltpu.VMEM((_W, 128), jnp.float32),
            pltpu.VMEM((_W,), jnp.int32),
            pltpu.VMEM_SHARED((n_pad, 128), jnp.float32),
        ])
    def run(m_hbm, di_hbm, z_hbm, acc_hbm, mb, dib, shared):
        cid = lax.axis_index("c")
        sid = lax.axis_index("s")
        wid = sid * 2 + cid
        base0 = wid * epw

        pltpu.sync_copy(z_hbm, shared.at[pl.ds(sid * npt, npt)])
        plsc.subcore_barrier()

        def win(w, carry):
            base = base0 + w * _W
            pltpu.sync_copy(m_hbm.at[pl.ds(base, _W)], mb)
            pltpu.sync_copy(di_hbm.at[pl.ds(base, _W)], dib)
            pltpu.sync_copy(mb, shared.at[dib], add=True)
            return carry

        lax.fori_loop(0, nwin, win, 0)
        plsc.subcore_barrier()
        pltpu.sync_copy(shared.at[pl.ds(sid * npt, npt)],
                        acc_hbm.at[cid].at[pl.ds(sid * npt, npt)])

    return run(m, dst, zeros)


def kernel(h, x, rel_x, r_feat, edge_feat, invar_ligand_shape,
           ligand_shape_emb, topo_out, e_w, params, edge_index):
    n, in_dim = h.shape
    e = rel_x.shape[0]
    pk, pv, pq = params['xk'], params['xv'], params['xq']

    w1k, w1v = pk['W1'], pv['W1']
    wkh, wkt, wki = w1k[32:160], w1k[288:416], w1k[416:448]
    wvh, wvt, wvi = w1v[32:160], w1v[288:416], w1v[416:448]
    wsrc = jnp.concatenate([w1k[160:288], w1v[160:288]], axis=1)
    wef = jnp.concatenate([w1k[0:32], w1v[0:32]], axis=1)
    b1k = pk['b1'][None, :]
    b1v = pv['b1'][None, :]
    s = jnp.kron(jnp.eye(16, dtype=jnp.float32),
                 jnp.ones((8, 1), jnp.float32)) * (1.0 / math.sqrt(8.0))

    src = edge_index[0]
    dst = edge_index[1]

    bn = 1000
    node_ws = [wkh, wkt, wki, b1k, wvh, wvt, wvi, b1v,
               pq['W1'], pq['b1'][None, :], pq['g'][None, :],
               pq['beta'][None, :], pq['W2'], pq['b2'][None, :]]
    tdst = pl.pallas_call(
        _node_body,
        grid=(n // bn,),
        in_specs=[pl.BlockSpec((bn, in_dim), lambda i: (i, 0)),
                  pl.BlockSpec((bn, 128), lambda i: (i, 0)),
                  pl.BlockSpec((bn, 32), lambda i: (i, 0))]
                 + [pl.BlockSpec(w.shape, lambda i, nd=w.ndim: (0,) * nd)
                    for w in node_ws],
        out_specs=pl.BlockSpec((bn, 384), lambda i: (i, 0)),
        out_shape=jax.ShapeDtypeStruct((n, 384), jnp.float32),
    )(h, topo_out, invar_ligand_shape, *node_ws)

    gdst, gh = _sc_gather(tdst, h, dst, src)

    be = 1000
    edge_ws = [wsrc, wef,
               pk['g'][None, :], pk['beta'][None, :], pk['W2'], pk['b2'][None, :],
               pv['g'][None, :], pv['beta'][None, :], pv['W2'], pv['b2'][None, :],
               s]
    m = pl.pallas_call(
        _edge_body,
        grid=(e // be,),
        in_specs=[pl.BlockSpec((be, 384), lambda i: (i, 0)),
                  pl.BlockSpec((be, 128), lambda i: (i, 0)),
                  pl.BlockSpec((be, 16), lambda i: (i, 0)),
                  pl.BlockSpec((be, 16), lambda i: (i, 0)),
                  pl.BlockSpec((be, 1), lambda i: (i, 0)),
                  pl.BlockSpec((be, 3), lambda i: (i, 0))]
                 + [pl.BlockSpec(w.shape, lambda i, nd=w.ndim: (0,) * nd)
                    for w in edge_ws],
        out_specs=pl.BlockSpec((be, 128), lambda i: (i, 0)),
        out_shape=jax.ShapeDtypeStruct((e, 128), jnp.float32),
    )(gdst, gh, edge_feat, r_feat, e_w[:, None], rel_x, *edge_ws)

    npt = ((n + 15) // 16 + 7) // 8 * 8
    zeros = jnp.zeros((npt, 128), jnp.float32)
    acc = _sc_scatter(m, dst, zeros, n)

    bf = 1000
    lse_t = jnp.transpose(ligand_shape_emb, (2, 0, 1))
    wf, wd = params['Wf'], params['Wd']
    head_ws = [wf[0:1], wf[1:17], wf[17:49], wd[0:1], wd[1:17], wd[17:49]]
    out = pl.pallas_call(
        _final_body,
        grid=(n // bf,),
        in_specs=[pl.BlockSpec((2, bf, 128), lambda i: (0, i, 0)),
                  pl.BlockSpec((bf, 3), lambda i: (i, 0)),
                  pl.BlockSpec((3, bf, 32), lambda i: (0, i, 0))]
                 + [pl.BlockSpec(w.shape, lambda i, nd=w.ndim: (0,) * nd)
                    for w in head_ws],
        out_specs=pl.BlockSpec((bf, 3), lambda i: (i, 0)),
        out_shape=jax.ShapeDtypeStruct((n, 3), jnp.float32),
    )(acc, x, lse_t, *head_ws)
    return out


# speedup vs baseline: 27.1378x; 1.0901x over previous
import functools
import math

import jax
import jax.numpy as jnp
from jax import lax
from jax.experimental import pallas as pl
from jax.experimental.pallas import tpu as pltpu
from jax.experimental.pallas import tpu_sc as plsc

_NW = 32
_W = 80
_LN_EPS = 1e-5
_VN_EPS = 1e-6


def _bf16_bits(x):
    r = x.astype(jnp.bfloat16).astype(jnp.float32)
    return lax.bitcast_convert_type(r, jnp.int32)


def _pack2(lo, hi):
    w = lax.shift_right_logical(_bf16_bits(lo), 16) | _bf16_bits(hi)
    return lax.bitcast_convert_type(w, jnp.float32)


def _unpack_lo(w):
    wi = lax.bitcast_convert_type(w, jnp.int32)
    return lax.bitcast_convert_type(lax.shift_left(wi, 16), jnp.float32)


def _unpack_hi(w):
    wi = lax.bitcast_convert_type(w, jnp.int32)
    return lax.bitcast_convert_type(wi & jnp.int32(-65536), jnp.float32)


def _node_body(h_ref, topo_ref, invar_ref,
               wkh_ref, wkt_ref, wki_ref, b1k_ref,
               wvh_ref, wvt_ref, wvi_ref, b1v_ref,
               w1q_ref, b1q_ref, gq_ref, betaq_ref, w2q_ref, b2q_ref,
               wsrc_ref, td_ref, ts_ref):
    h = h_ref[...]
    topo = topo_ref[...]
    invar = invar_ref[...]
    adk = h @ wkh_ref[...] + topo @ wkt_ref[...] + invar @ wki_ref[...] + b1k_ref[...]
    adv = h @ wvh_ref[...] + topo @ wvt_ref[...] + invar @ wvi_ref[...] + b1v_ref[...]
    y = h @ w1q_ref[...] + b1q_ref[...]
    mu = jnp.mean(y, axis=-1, keepdims=True)
    var = jnp.mean((y - mu) ** 2, axis=-1, keepdims=True)
    z = (y - mu) * lax.rsqrt(var + _LN_EPS) * gq_ref[...] + betaq_ref[...]
    z = jnp.maximum(z, 0.0)
    q = z @ w2q_ref[...] + b2q_ref[...]
    td_ref[...] = _pack2(jnp.concatenate([adk, adv], axis=1),
                         jnp.concatenate([q, jnp.zeros_like(q)], axis=1))
    asrc = h @ wsrc_ref[...]
    ts_ref[...] = _pack2(asrc[:, 0:128], asrc[:, 128:256])


def _ln_relu(y, g, beta):
    mu = jnp.mean(y, axis=-1, keepdims=True)
    var = jnp.mean((y - mu) ** 2, axis=-1, keepdims=True)
    z = (y - mu) * lax.rsqrt(var + _LN_EPS) * g + beta
    return jnp.maximum(z, 0.0)


def _edge_body(gd_ref, gh_ref, ef_ref, rf_ref, ew_ref, rx_ref,
               wef_ref,
               gk_ref, betak_ref, w2k_ref, b2k_ref,
               gv_ref, betav_ref, w2v_ref, b2v_ref, s_ref,
               out_ref):
    wd = gd_ref[...]
    ws = gh_ref[...]
    ad = _unpack_lo(wd)
    qg = _unpack_hi(wd)[:, 0:128]
    efrf = jnp.concatenate([ef_ref[...], rf_ref[...]], axis=1)
    ep = efrf @ wef_ref[...]
    y1k = ad[:, 0:128] + _unpack_lo(ws) + ep[:, 0:128]
    y1v = ad[:, 128:256] + _unpack_hi(ws) + ep[:, 128:256]
    zk = _ln_relu(y1k, gk_ref[...], betak_ref[...])
    zv = _ln_relu(y1v, gv_ref[...], betav_ref[...])
    kk = zk @ w2k_ref[...] + b2k_ref[...]
    vs = (zv @ w2v_ref[...] + b2v_ref[...]) * ew_ref[...]
    ex = jnp.exp((qg * kk) @ s_ref[...])
    w = ex * vs
    rx = rx_ref[...]
    pad = jnp.zeros((ex.shape[0], 64), jnp.float32)
    out_ref[...] = jnp.concatenate(
        [ex, w * rx[:, 0:1], w * rx[:, 1:2], w * rx[:, 2:3], pad], axis=1)


def _final_body(acc_ref, x_ref, lse_ref,
                wf0_ref, wfo_ref, wfl_ref, wd0_ref, wdo_ref, wdl_ref,
                out_ref):
    a = acc_ref[0] + acc_ref[1]
    denom = a[:, 0:16] + 1e-16
    ods, ps, ds_ = [], [], []
    for d in range(3):
        od = a[:, 16 * (d + 1):16 * (d + 2)] / denom
        xd = x_ref[:, d:d + 1]
        ld = lse_ref[d]
        ods.append(od)
        ps.append(xd * wf0_ref[...] + od @ wfo_ref[...] + ld @ wfl_ref[...])
        ds_.append(xd * wd0_ref[...] + od @ wdo_ref[...] + ld @ wdl_ref[...])
    dot = ps[0] * ds_[0] + ps[1] * ds_[1] + ps[2] * ds_[2]
    dns = ds_[0] * ds_[0] + ds_[1] * ds_[1] + ds_[2] * ds_[2]
    coef = dot / (dns + _VN_EPS)
    fneg = jnp.where(dot >= 0, 0.0, 0.8)
    cols = []
    for d in range(3):
        vn = ps[d] - fneg * coef * ds_[d]
        cols.append(jnp.mean(ods[d], axis=1, keepdims=True)
                    + jnp.mean(vn, axis=1, keepdims=True))
    out_ref[...] = jnp.concatenate(cols, axis=1)


def _sc_gather(tdst, hsrc, dst, src):
    e = dst.shape[0]
    epw = e // _NW
    nwin = epw // _W
    dt = tdst.shape[1]
    dh = hsrc.shape[1]
    mesh = plsc.VectorSubcoreMesh(core_axis_name="c", subcore_axis_name="s")

    @functools.partial(
        pl.kernel,
        out_type=[jax.ShapeDtypeStruct((e, dt), jnp.float32),
                  jax.ShapeDtypeStruct((e, dh), jnp.float32)],
        mesh=mesh,
        scratch_types=[
            pltpu.VMEM((_W,), jnp.int32),
            pltpu.VMEM((_W,), jnp.int32),
            pltpu.VMEM((_W, dt), jnp.float32),
            pltpu.VMEM((_W, dh), jnp.float32),
            pltpu.SemaphoreType.DMA,
            pltpu.SemaphoreType.DMA,
        ])
    def run(tdst_hbm, h_hbm, di_hbm, si_hbm, gd_hbm, gh_hbm,
            dib, sib, drows, hrows, sem_d, sem_s):
        cid = lax.axis_index("c")
        sid = lax.axis_index("s")
        wid = sid * 2 + cid
        base0 = wid * epw

        def win(w, carry):
            base = base0 + w * _W
            pltpu.sync_copy(di_hbm.at[pl.ds(base, _W)], dib)
            pltpu.sync_copy(si_hbm.at[pl.ds(base, _W)], sib)
            cp1 = pltpu.async_copy(tdst_hbm.at[dib], drows, sem_d)
            cp2 = pltpu.async_copy(h_hbm.at[sib], hrows, sem_s)
            cp1.wait()
            cp2.wait()
            pltpu.sync_copy(drows, gd_hbm.at[pl.ds(base, _W)])
            pltpu.sync_copy(hrows, gh_hbm.at[pl.ds(base, _W)])
            return carry

        lax.fori_loop(0, nwin, win, 0)

    return run(tdst, hsrc, dst, src)


def _sc_scatter(m, dst, zeros, n):
    e = dst.shape[0]
    epw = e // _NW
    nwin = epw // _W
    npt = zeros.shape[0]
    n_pad = npt * 16
    mesh = plsc.VectorSubcoreMesh(core_axis_name="c", subcore_axis_name="s")

    @functools.partial(
        pl.kernel,
        out_type=jax.ShapeDtypeStruct((2, n_pad, 128), jnp.float32),
        mesh=mesh,
        scratch_types=[
            pltpu.VMEM((_W, 128), jnp.float32),
            pltpu.VMEM((_W,), jnp.int32),
            pltpu.VMEM_SHARED((n_pad, 128), jnp.float32),
        ])
    def run(m_hbm, di_hbm, z_hbm, acc_hbm, mb, dib, shared):
        cid = lax.axis_index("c")
        sid = lax.axis_index("s")
        wid = sid * 2 + cid
        base0 = wid * epw

        pltpu.sync_copy(z_hbm, shared.at[pl.ds(sid * npt, npt)])
        plsc.subcore_barrier()

        def win(w, carry):
            base = base0 + w * _W
            pltpu.sync_copy(m_hbm.at[pl.ds(base, _W)], mb)
            pltpu.sync_copy(di_hbm.at[pl.ds(base, _W)], dib)
            pltpu.sync_copy(mb, shared.at[dib], add=True)
            return carry

        lax.fori_loop(0, nwin, win, 0)
        plsc.subcore_barrier()
        pltpu.sync_copy(shared.at[pl.ds(sid * npt, npt)],
                        acc_hbm.at[cid].at[pl.ds(sid * npt, npt)])

    return run(m, dst, zeros)


def kernel(h, x, rel_x, r_feat, edge_feat, invar_ligand_shape,
           ligand_shape_emb, topo_out, e_w, params, edge_index):
    n, in_dim = h.shape
    e = rel_x.shape[0]
    pk, pv, pq = params['xk'], params['xv'], params['xq']

    w1k, w1v = pk['W1'], pv['W1']
    wkh, wkt, wki = w1k[32:160], w1k[288:416], w1k[416:448]
    wvh, wvt, wvi = w1v[32:160], w1v[288:416], w1v[416:448]
    wsrc = jnp.concatenate([w1k[160:288], w1v[160:288]], axis=1)
    wef = jnp.concatenate([w1k[0:32], w1v[0:32]], axis=1)
    b1k = pk['b1'][None, :]
    b1v = pv['b1'][None, :]
    s = jnp.kron(jnp.eye(16, dtype=jnp.float32),
                 jnp.ones((8, 1), jnp.float32)) * (1.0 / math.sqrt(8.0))

    src = edge_index[0]
    dst = edge_index[1]

    bn = 1000
    node_ws = [wkh, wkt, wki, b1k, wvh, wvt, wvi, b1v,
               pq['W1'], pq['b1'][None, :], pq['g'][None, :],
               pq['beta'][None, :], pq['W2'], pq['b2'][None, :], wsrc]
    tdst, tsrc = pl.pallas_call(
        _node_body,
        grid=(n // bn,),
        in_specs=[pl.BlockSpec((bn, in_dim), lambda i: (i, 0)),
                  pl.BlockSpec((bn, 128), lambda i: (i, 0)),
                  pl.BlockSpec((bn, 32), lambda i: (i, 0))]
                 + [pl.BlockSpec(w.shape, lambda i, nd=w.ndim: (0,) * nd)
                    for w in node_ws],
        out_specs=[pl.BlockSpec((bn, 256), lambda i: (i, 0)),
                   pl.BlockSpec((bn, 128), lambda i: (i, 0))],
        out_shape=[jax.ShapeDtypeStruct((n, 256), jnp.float32),
                   jax.ShapeDtypeStruct((n, 128), jnp.float32)],
    )(h, topo_out, invar_ligand_shape, *node_ws)

    gdst, gh = _sc_gather(tdst, tsrc, dst, src)

    be = 1000
    edge_ws = [wef,
               pk['g'][None, :], pk['beta'][None, :], pk['W2'], pk['b2'][None, :],
               pv['g'][None, :], pv['beta'][None, :], pv['W2'], pv['b2'][None, :],
               s]
    m = pl.pallas_call(
        _edge_body,
        grid=(e // be,),
        in_specs=[pl.BlockSpec((be, 256), lambda i: (i, 0)),
                  pl.BlockSpec((be, 128), lambda i: (i, 0)),
                  pl.BlockSpec((be, 16), lambda i: (i, 0)),
                  pl.BlockSpec((be, 16), lambda i: (i, 0)),
                  pl.BlockSpec((be, 1), lambda i: (i, 0)),
                  pl.BlockSpec((be, 3), lambda i: (i, 0))]
                 + [pl.BlockSpec(w.shape, lambda i, nd=w.ndim: (0,) * nd)
                    for w in edge_ws],
        out_specs=pl.BlockSpec((be, 128), lambda i: (i, 0)),
        out_shape=jax.ShapeDtypeStruct((e, 128), jnp.float32),
    )(gdst, gh, edge_feat, r_feat, e_w[:, None], rel_x, *edge_ws)

    npt = ((n + 15) // 16 + 7) // 8 * 8
    zeros = jnp.zeros((npt, 128), jnp.float32)
    acc = _sc_scatter(m, dst, zeros, n)

    bf = 1000
    lse_t = jnp.transpose(ligand_shape_emb, (2, 0, 1))
    wf, wd = params['Wf'], params['Wd']
    head_ws = [wf[0:1], wf[1:17], wf[17:49], wd[0:1], wd[1:17], wd[17:49]]
    out = pl.pallas_call(
        _final_body,
        grid=(n // bf,),
        in_specs=[pl.BlockSpec((2, bf, 128), lambda i: (0, i, 0)),
                  pl.BlockSpec((bf, 3), lambda i: (i, 0)),
                  pl.BlockSpec((3, bf, 32), lambda i: (0, i, 0))]
                 + [pl.BlockSpec(w.shape, lambda i, nd=w.ndim: (0,) * nd)
                    for w in head_ws],
        out_specs=pl.BlockSpec((bf, 3), lambda i: (i, 0)),
        out_shape=jax.ShapeDtypeStruct((n, 3), jnp.float32),
    )(acc, x, lse_t, *head_ws)
    return out


# speedup vs baseline: 31.3273x; 1.1544x over previous
import functools
import math

import jax
import jax.numpy as jnp
from jax import lax
from jax.experimental import pallas as pl
from jax.experimental.pallas import tpu as pltpu
from jax.experimental.pallas import tpu_sc as plsc

_NW = 32
_W = 80
_LN_EPS = 1e-5
_VN_EPS = 1e-6


def _bf16_bits(x):
    r = x.astype(jnp.bfloat16).astype(jnp.float32)
    return lax.bitcast_convert_type(r, jnp.int32)


def _pack2(lo, hi):
    w = lax.shift_right_logical(_bf16_bits(lo), 16) | _bf16_bits(hi)
    return lax.bitcast_convert_type(w, jnp.float32)


def _unpack_lo(w):
    wi = lax.bitcast_convert_type(w, jnp.int32)
    return lax.bitcast_convert_type(lax.shift_left(wi, 16), jnp.float32)


def _unpack_hi(w):
    wi = lax.bitcast_convert_type(w, jnp.int32)
    return lax.bitcast_convert_type(wi & jnp.int32(-65536), jnp.float32)


def _node_body(h_ref, topo_ref, invar_ref,
               wkh_ref, wkt_ref, wki_ref, b1k_ref,
               wvh_ref, wvt_ref, wvi_ref, b1v_ref,
               w1q_ref, b1q_ref, gq_ref, betaq_ref, w2q_ref, b2q_ref,
               wsrc_ref, td_ref, ts_ref):
    h = h_ref[...]
    topo = topo_ref[...]
    invar = invar_ref[...]
    adk = h @ wkh_ref[...] + topo @ wkt_ref[...] + invar @ wki_ref[...] + b1k_ref[...]
    adv = h @ wvh_ref[...] + topo @ wvt_ref[...] + invar @ wvi_ref[...] + b1v_ref[...]
    y = h @ w1q_ref[...] + b1q_ref[...]
    mu = jnp.mean(y, axis=-1, keepdims=True)
    var = jnp.mean((y - mu) ** 2, axis=-1, keepdims=True)
    z = (y - mu) * lax.rsqrt(var + _LN_EPS) * gq_ref[...] + betaq_ref[...]
    z = jnp.maximum(z, 0.0)
    q = z @ w2q_ref[...] + b2q_ref[...]
    td_ref[...] = _pack2(jnp.concatenate([adk, adv], axis=1),
                         jnp.concatenate([q, jnp.zeros_like(q)], axis=1))
    asrc = h @ wsrc_ref[...]
    ts_ref[...] = _pack2(asrc[:, 0:128], asrc[:, 128:256])


def _ln_relu(y, g, beta):
    mu = jnp.mean(y, axis=-1, keepdims=True)
    var = jnp.mean((y - mu) ** 2, axis=-1, keepdims=True)
    z = (y - mu) * lax.rsqrt(var + _LN_EPS) * g + beta
    return jnp.maximum(z, 0.0)


def _edge_body(gd_ref, gh_ref, ef_ref, rf_ref, ew_ref, rx_ref,
               wef_ref,
               gk_ref, betak_ref, w2k_ref, b2k_ref,
               gv_ref, betav_ref, w2v_ref, b2v_ref, s_ref,
               out_ref):
    wd = gd_ref[...]
    ws = gh_ref[...]
    ad = _unpack_lo(wd)
    qg = _unpack_hi(wd)[:, 0:128]
    efrf = jnp.concatenate([ef_ref[...], rf_ref[...]], axis=1)
    ep = efrf @ wef_ref[...]
    y1k = ad[:, 0:128] + _unpack_lo(ws) + ep[:, 0:128]
    y1v = ad[:, 128:256] + _unpack_hi(ws) + ep[:, 128:256]
    zk = _ln_relu(y1k, gk_ref[...], betak_ref[...])
    zv = _ln_relu(y1v, gv_ref[...], betav_ref[...])
    kk = zk @ w2k_ref[...] + b2k_ref[...]
    vs = (zv @ w2v_ref[...] + b2v_ref[...]) * ew_ref[...]
    ex = jnp.exp((qg * kk) @ s_ref[...])
    w = ex * vs
    rx = rx_ref[...]
    pad = jnp.zeros((ex.shape[0], 64), jnp.float32)
    out_ref[...] = jnp.concatenate(
        [ex, w * rx[:, 0:1], w * rx[:, 1:2], w * rx[:, 2:3], pad], axis=1)


def _final_body(acc_ref, x_ref, lse_ref,
                wf0_ref, wfo_ref, wfl_ref, wd0_ref, wdo_ref, wdl_ref,
                out_ref):
    a = acc_ref[0] + acc_ref[1]
    denom = a[:, 0:16] + 1e-16
    ods, ps, ds_ = [], [], []
    for d in range(3):
        od = a[:, 16 * (d + 1):16 * (d + 2)] / denom
        xd = x_ref[:, d:d + 1]
        ld = lse_ref[d]
        ods.append(od)
        ps.append(xd * wf0_ref[...] + od @ wfo_ref[...] + ld @ wfl_ref[...])
        ds_.append(xd * wd0_ref[...] + od @ wdo_ref[...] + ld @ wdl_ref[...])
    dot = ps[0] * ds_[0] + ps[1] * ds_[1] + ps[2] * ds_[2]
    dns = ds_[0] * ds_[0] + ds_[1] * ds_[1] + ds_[2] * ds_[2]
    coef = dot / (dns + _VN_EPS)
    fneg = jnp.where(dot >= 0, 0.0, 0.8)
    cols = []
    for d in range(3):
        vn = ps[d] - fneg * coef * ds_[d]
        cols.append(jnp.mean(ods[d], axis=1, keepdims=True)
                    + jnp.mean(vn, axis=1, keepdims=True))
    out_ref[...] = jnp.concatenate(cols, axis=1)


def _sc_gather(tdst, hsrc, dst, src):
    e = dst.shape[0]
    epw = e // _NW
    wb = 128
    nfull = epw // wb
    tail = epw - nfull * wb
    npair = nfull // 2
    odd = nfull % 2
    dt = tdst.shape[1]
    dh = hsrc.shape[1]
    mesh = plsc.VectorSubcoreMesh(core_axis_name="c", subcore_axis_name="s")

    @functools.partial(
        pl.kernel,
        out_type=[jax.ShapeDtypeStruct((e, dt), jnp.float32),
                  jax.ShapeDtypeStruct((e, dh), jnp.float32)],
        mesh=mesh,
        scratch_types=[
            pltpu.VMEM((epw,), jnp.int32),
            pltpu.VMEM((epw,), jnp.int32),
            pltpu.VMEM((2, wb, dt), jnp.float32),
            pltpu.VMEM((2, wb, dh), jnp.float32),
            pltpu.SemaphoreType.DMA,
            pltpu.SemaphoreType.DMA,
            pltpu.SemaphoreType.DMA,
            pltpu.SemaphoreType.DMA,
        ])
    def run(tdst_hbm, h_hbm, di_hbm, si_hbm, gd_hbm, gh_hbm,
            dia, sia, drows, hrows, sgd, sgh, swd, swh):
        cid = lax.axis_index("c")
        sid = lax.axis_index("s")
        wid = sid * 2 + cid
        base0 = wid * epw
        pltpu.sync_copy(di_hbm.at[pl.ds(base0, epw)], dia)
        pltpu.sync_copy(si_hbm.at[pl.ds(base0, epw)], sia)

        def one(w, b, drain):
            base = base0 + w * wb
            off = w * wb
            if drain is not None:
                @pl.when(drain)
                def _():
                    pltpu.make_async_copy(
                        drows.at[b], gd_hbm.at[pl.ds(base0, wb)], swd).wait()
                    pltpu.make_async_copy(
                        hrows.at[b], gh_hbm.at[pl.ds(base0, wb)], swh).wait()
            cp1 = pltpu.async_copy(
                tdst_hbm.at[dia.at[pl.ds(off, wb)]], drows.at[b], sgd)
            cp2 = pltpu.async_copy(
                h_hbm.at[sia.at[pl.ds(off, wb)]], hrows.at[b], sgh)
            cp1.wait()
            cp2.wait()
            pltpu.async_copy(drows.at[b], gd_hbm.at[pl.ds(base, wb)], swd)
            pltpu.async_copy(hrows.at[b], gh_hbm.at[pl.ds(base, wb)], swh)

        def pair(g, carry):
            for b in range(2):
                one(2 * g + b, b, g >= 1)
            return carry

        lax.fori_loop(0, npair, pair, 0)
        if odd:
            one(nfull - 1, 0, npair >= 1)
        for b in range(2):
            pltpu.make_async_copy(
                drows.at[b], gd_hbm.at[pl.ds(base0, wb)], swd).wait()
            pltpu.make_async_copy(
                hrows.at[b], gh_hbm.at[pl.ds(base0, wb)], swh).wait()
        if tail:
            toff = nfull * wb
            tbase = base0 + toff
            cp1 = pltpu.async_copy(
                tdst_hbm.at[dia.at[pl.ds(toff, tail)]],
                drows.at[0].at[pl.ds(0, tail)], sgd)
            cp2 = pltpu.async_copy(
                h_hbm.at[sia.at[pl.ds(toff, tail)]],
                hrows.at[0].at[pl.ds(0, tail)], sgh)
            cp1.wait()
            cp2.wait()
            pltpu.sync_copy(drows.at[0].at[pl.ds(0, tail)],
                            gd_hbm.at[pl.ds(tbase, tail)])
            pltpu.sync_copy(hrows.at[0].at[pl.ds(0, tail)],
                            gh_hbm.at[pl.ds(tbase, tail)])

    return run(tdst, hsrc, dst, src)


def _sc_scatter(m, dst, zeros, n):
    e = dst.shape[0]
    epw = e // _NW
    wb = 128
    nfull = epw // wb
    tail = epw - nfull * wb
    npair = nfull // 2
    odd = nfull % 2
    npt = zeros.shape[0]
    n_pad = npt * 16
    mesh = plsc.VectorSubcoreMesh(core_axis_name="c", subcore_axis_name="s")

    @functools.partial(
        pl.kernel,
        out_type=jax.ShapeDtypeStruct((2, n_pad, 128), jnp.float32),
        mesh=mesh,
        scratch_types=[
            pltpu.VMEM((2, wb, 128), jnp.float32),
            pltpu.VMEM((2, wb), jnp.int32),
            pltpu.VMEM((max(tail, 8), 128), jnp.float32),
            pltpu.VMEM((max(tail, 8),), jnp.int32),
            pltpu.VMEM_SHARED((n_pad, 128), jnp.float32),
            pltpu.SemaphoreType.DMA,
            pltpu.SemaphoreType.DMA,
            pltpu.SemaphoreType.DMA,
        ])
    def run(m_hbm, di_hbm, z_hbm, acc_hbm, mb, dib, mtl, dtl, shared,
            slm, sli, ssc):
        cid = lax.axis_index("c")
        sid = lax.axis_index("s")
        wid = sid * 2 + cid
        base0 = wid * epw

        pltpu.sync_copy(z_hbm, shared.at[pl.ds(sid * npt, npt)])
        plsc.subcore_barrier()

        def one(w, b, drain):
            base = base0 + w * wb
            if drain is not None:
                @pl.when(drain)
                def _():
                    pltpu.make_async_copy(
                        mb.at[b], shared.at[pl.ds(0, wb)], ssc).wait()
            cp1 = pltpu.async_copy(m_hbm.at[pl.ds(base, wb)], mb.at[b], slm)
            cp2 = pltpu.async_copy(di_hbm.at[pl.ds(base, wb)], dib.at[b], sli)
            cp1.wait()
            cp2.wait()
            pltpu.async_copy(mb.at[b], shared.at[dib.at[b]], ssc, add=True)

        def pair(g, carry):
            for b in range(2):
                one(2 * g + b, b, g >= 1)
            return carry

        lax.fori_loop(0, npair, pair, 0)
        if odd:
            one(nfull - 1, 0, npair >= 1)
        for b in range(2):
            pltpu.make_async_copy(
                mb.at[b], shared.at[pl.ds(0, wb)], ssc).wait()
        if tail:
            tbase = base0 + nfull * wb
            pltpu.sync_copy(m_hbm.at[pl.ds(tbase, tail)], mtl)
            pltpu.sync_copy(di_hbm.at[pl.ds(tbase, tail)], dtl)
            pltpu.sync_copy(mtl, shared.at[dtl], add=True)
        plsc.subcore_barrier()
        pltpu.sync_copy(shared.at[pl.ds(sid * npt, npt)],
                        acc_hbm.at[cid].at[pl.ds(sid * npt, npt)])

    return run(m, dst, zeros)


def kernel(h, x, rel_x, r_feat, edge_feat, invar_ligand_shape,
           ligand_shape_emb, topo_out, e_w, params, edge_index):
    n, in_dim = h.shape
    e = rel_x.shape[0]
    pk, pv, pq = params['xk'], params['xv'], params['xq']

    w1k, w1v = pk['W1'], pv['W1']
    wkh, wkt, wki = w1k[32:160], w1k[288:416], w1k[416:448]
    wvh, wvt, wvi = w1v[32:160], w1v[288:416], w1v[416:448]
    wsrc = jnp.concatenate([w1k[160:288], w1v[160:288]], axis=1)
    wef = jnp.concatenate([w1k[0:32], w1v[0:32]], axis=1)
    b1k = pk['b1'][None, :]
    b1v = pv['b1'][None, :]
    s = jnp.kron(jnp.eye(16, dtype=jnp.float32),
                 jnp.ones((8, 1), jnp.float32)) * (1.0 / math.sqrt(8.0))

    src = edge_index[0]
    dst = edge_index[1]

    bn = 1000
    node_ws = [wkh, wkt, wki, b1k, wvh, wvt, wvi, b1v,
               pq['W1'], pq['b1'][None, :], pq['g'][None, :],
               pq['beta'][None, :], pq['W2'], pq['b2'][None, :], wsrc]
    tdst, tsrc = pl.pallas_call(
        _node_body,
        grid=(n // bn,),
        in_specs=[pl.BlockSpec((bn, in_dim), lambda i: (i, 0)),
                  pl.BlockSpec((bn, 128), lambda i: (i, 0)),
                  pl.BlockSpec((bn, 32), lambda i: (i, 0))]
                 + [pl.BlockSpec(w.shape, lambda i, nd=w.ndim: (0,) * nd)
                    for w in node_ws],
        out_specs=[pl.BlockSpec((bn, 256), lambda i: (i, 0)),
                   pl.BlockSpec((bn, 128), lambda i: (i, 0))],
        out_shape=[jax.ShapeDtypeStruct((n, 256), jnp.float32),
                   jax.ShapeDtypeStruct((n, 128), jnp.float32)],
    )(h, topo_out, invar_ligand_shape, *node_ws)

    gdst, gh = _sc_gather(tdst, tsrc, dst, src)

    be = 2000
    edge_ws = [wef,
               pk['g'][None, :], pk['beta'][None, :], pk['W2'], pk['b2'][None, :],
               pv['g'][None, :], pv['beta'][None, :], pv['W2'], pv['b2'][None, :],
               s]
    m = pl.pallas_call(
        _edge_body,
        grid=(e // be,),
        in_specs=[pl.BlockSpec((be, 256), lambda i: (i, 0)),
                  pl.BlockSpec((be, 128), lambda i: (i, 0)),
                  pl.BlockSpec((be, 16), lambda i: (i, 0)),
                  pl.BlockSpec((be, 16), lambda i: (i, 0)),
                  pl.BlockSpec((be, 1), lambda i: (i, 0)),
                  pl.BlockSpec((be, 3), lambda i: (i, 0))]
                 + [pl.BlockSpec(w.shape, lambda i, nd=w.ndim: (0,) * nd)
                    for w in edge_ws],
        out_specs=pl.BlockSpec((be, 128), lambda i: (i, 0)),
        out_shape=jax.ShapeDtypeStruct((e, 128), jnp.float32),
    )(gdst, gh, edge_feat, r_feat, e_w[:, None], rel_x, *edge_ws)

    npt = ((n + 15) // 16 + 7) // 8 * 8
    zeros = jnp.zeros((npt, 128), jnp.float32)
    acc = _sc_scatter(m, dst, zeros, n)

    bf = 1000
    lse_t = jnp.transpose(ligand_shape_emb, (2, 0, 1))
    wf, wd = params['Wf'], params['Wd']
    head_ws = [wf[0:1], wf[1:17], wf[17:49], wd[0:1], wd[1:17], wd[17:49]]
    out = pl.pallas_call(
        _final_body,
        grid=(n // bf,),
        in_specs=[pl.BlockSpec((2, bf, 128), lambda i: (0, i, 0)),
                  pl.BlockSpec((bf, 3), lambda i: (i, 0)),
                  pl.BlockSpec((3, bf, 32), lambda i: (0, i, 0))]
                 + [pl.BlockSpec(w.shape, lambda i, nd=w.ndim: (0,) * nd)
                    for w in head_ws],
        out_specs=pl.BlockSpec((bf, 3), lambda i: (i, 0)),
        out_shape=jax.ShapeDtypeStruct((n, 3), jnp.float32),
    )(acc, x, lse_t, *head_ws)
    return out


# speedup vs baseline: 33.6921x; 1.0755x over previous
import functools
import math

import jax
import jax.numpy as jnp
from jax import lax
from jax.experimental import pallas as pl
from jax.experimental.pallas import tpu as pltpu
from jax.experimental.pallas import tpu_sc as plsc

_NW = 32
_W = 80
_LN_EPS = 1e-5
_VN_EPS = 1e-6


def _bf16_bits(x):
    r = x.astype(jnp.bfloat16).astype(jnp.float32)
    return lax.bitcast_convert_type(r, jnp.int32)


def _pack2(lo, hi):
    w = lax.shift_right_logical(_bf16_bits(lo), 16) | _bf16_bits(hi)
    return lax.bitcast_convert_type(w, jnp.float32)


def _unpack_lo(w):
    wi = lax.bitcast_convert_type(w, jnp.int32)
    return lax.bitcast_convert_type(lax.shift_left(wi, 16), jnp.float32)


def _unpack_hi(w):
    wi = lax.bitcast_convert_type(w, jnp.int32)
    return lax.bitcast_convert_type(wi & jnp.int32(-65536), jnp.float32)


def _node_body(h_ref, topo_ref, invar_ref,
               wkh_ref, wkt_ref, wki_ref, b1k_ref,
               wvh_ref, wvt_ref, wvi_ref, b1v_ref,
               w1q_ref, b1q_ref, gq_ref, betaq_ref, w2q_ref, b2q_ref,
               wsrc_ref, td_ref, ts_ref):
    h = h_ref[...]
    topo = topo_ref[...]
    invar = invar_ref[...]
    adk = h @ wkh_ref[...] + topo @ wkt_ref[...] + invar @ wki_ref[...] + b1k_ref[...]
    adv = h @ wvh_ref[...] + topo @ wvt_ref[...] + invar @ wvi_ref[...] + b1v_ref[...]
    y = h @ w1q_ref[...] + b1q_ref[...]
    mu = jnp.mean(y, axis=-1, keepdims=True)
    var = jnp.mean((y - mu) ** 2, axis=-1, keepdims=True)
    z = (y - mu) * lax.rsqrt(var + _LN_EPS) * gq_ref[...] + betaq_ref[...]
    z = jnp.maximum(z, 0.0)
    q = z @ w2q_ref[...] + b2q_ref[...]
    td_ref[...] = _pack2(jnp.concatenate([adk, adv], axis=1),
                         jnp.concatenate([q, jnp.zeros_like(q)], axis=1))
    asrc = h @ wsrc_ref[...]
    ts_ref[...] = _pack2(asrc[:, 0:128], asrc[:, 128:256])


def _ln_relu(y, g, beta):
    mu = jnp.mean(y, axis=-1, keepdims=True)
    var = jnp.mean((y - mu) ** 2, axis=-1, keepdims=True)
    z = (y - mu) * lax.rsqrt(var + _LN_EPS) * g + beta
    return jnp.maximum(z, 0.0)


def _edge_body(gd_ref, gh_ref, ef_ref, rf_ref, ew_ref, rx_ref,
               wef_ref,
               gk_ref, betak_ref, w2k_ref, b2k_ref,
               gv_ref, betav_ref, w2vr_ref, b2vr_ref, sr_ref, rmat_ref,
               out_ref):
    wd = gd_ref[...]
    ws = gh_ref[...]
    ad = _unpack_lo(wd)
    qg = _unpack_hi(wd)[:, 0:128]
    efrf = jnp.concatenate([ef_ref[...], rf_ref[...]], axis=1)
    ep = efrf @ wef_ref[...]
    y1k = ad[:, 0:128] + _unpack_lo(ws) + ep[:, 0:128]
    y1v = ad[:, 128:256] + _unpack_hi(ws) + ep[:, 128:256]
    zk = _ln_relu(y1k, gk_ref[...], betak_ref[...])
    zv = _ln_relu(y1v, gv_ref[...], betav_ref[...])
    kk = zk @ w2k_ref[...] + b2k_ref[...]
    exr = jnp.exp((qg * kk) @ sr_ref[...])
    vsr = zv @ w2vr_ref[...] + b2vr_ref[...]
    rxin = jnp.concatenate(
        [jnp.ones_like(ew_ref[...]), rx_ref[...] * ew_ref[...]], axis=1)
    out_ref[...] = exr * vsr * (rxin @ rmat_ref[...])


def _final_body(acc_ref, x_ref, lse_ref,
                wf0_ref, wfo_ref, wfl_ref, wd0_ref, wdo_ref, wdl_ref,
                out_ref):
    a = acc_ref[0] + acc_ref[1]
    denom = a[:, 0:16] + 1e-16
    ods, ps, ds_ = [], [], []
    for d in range(3):
        od = a[:, 16 * (d + 1):16 * (d + 2)] / denom
        xd = x_ref[:, d:d + 1]
        ld = lse_ref[d]
        ods.append(od)
        ps.append(xd * wf0_ref[...] + od @ wfo_ref[...] + ld @ wfl_ref[...])
        ds_.append(xd * wd0_ref[...] + od @ wdo_ref[...] + ld @ wdl_ref[...])
    dot = ps[0] * ds_[0] + ps[1] * ds_[1] + ps[2] * ds_[2]
    dns = ds_[0] * ds_[0] + ds_[1] * ds_[1] + ds_[2] * ds_[2]
    coef = dot / (dns + _VN_EPS)
    fneg = jnp.where(dot >= 0, 0.0, 0.8)
    cols = []
    for d in range(3):
        vn = ps[d] - fneg * coef * ds_[d]
        cols.append(jnp.mean(ods[d], axis=1, keepdims=True)
                    + jnp.mean(vn, axis=1, keepdims=True))
    out_ref[...] = jnp.concatenate(cols, axis=1)


def _sc_gather(tdst, hsrc, dst, src):
    e = dst.shape[0]
    epw = e // _NW
    wb = 128
    nfull = epw // wb
    tail = epw - nfull * wb
    npair = nfull // 2
    odd = nfull % 2
    dt = tdst.shape[1]
    dh = hsrc.shape[1]
    mesh = plsc.VectorSubcoreMesh(core_axis_name="c", subcore_axis_name="s")

    @functools.partial(
        pl.kernel,
        out_type=[jax.ShapeDtypeStruct((e, dt), jnp.float32),
                  jax.ShapeDtypeStruct((e, dh), jnp.float32)],
        mesh=mesh,
        scratch_types=[
            pltpu.VMEM((epw,), jnp.int32),
            pltpu.VMEM((epw,), jnp.int32),
            pltpu.VMEM((2, wb, dt), jnp.float32),
            pltpu.VMEM((2, wb, dh), jnp.float32),
            pltpu.SemaphoreType.DMA,
            pltpu.SemaphoreType.DMA,
            pltpu.SemaphoreType.DMA,
            pltpu.SemaphoreType.DMA,
        ])
    def run(tdst_hbm, h_hbm, di_hbm, si_hbm, gd_hbm, gh_hbm,
            dia, sia, drows, hrows, sgd, sgh, swd, swh):
        cid = lax.axis_index("c")
        sid = lax.axis_index("s")
        wid = sid * 2 + cid
        base0 = wid * epw
        pltpu.sync_copy(di_hbm.at[pl.ds(base0, epw)], dia)
        pltpu.sync_copy(si_hbm.at[pl.ds(base0, epw)], sia)

        def one(w, b, drain):
            base = base0 + w * wb
            off = w * wb
            if drain is not None:
                @pl.when(drain)
                def _():
                    pltpu.make_async_copy(
                        drows.at[b], gd_hbm.at[pl.ds(base0, wb)], swd).wait()
                    pltpu.make_async_copy(
                        hrows.at[b], gh_hbm.at[pl.ds(base0, wb)], swh).wait()
            cp1 = pltpu.async_copy(
                tdst_hbm.at[dia.at[pl.ds(off, wb)]], drows.at[b], sgd)
            cp2 = pltpu.async_copy(
                h_hbm.at[sia.at[pl.ds(off, wb)]], hrows.at[b], sgh)
            cp1.wait()
            cp2.wait()
            pltpu.async_copy(drows.at[b], gd_hbm.at[pl.ds(base, wb)], swd)
            pltpu.async_copy(hrows.at[b], gh_hbm.at[pl.ds(base, wb)], swh)

        def pair(g, carry):
            for b in range(2):
                one(2 * g + b, b, g >= 1)
            return carry

        lax.fori_loop(0, npair, pair, 0)
        if odd:
            one(nfull - 1, 0, npair >= 1)
        for b in range(2):
            pltpu.make_async_copy(
                drows.at[b], gd_hbm.at[pl.ds(base0, wb)], swd).wait()
            pltpu.make_async_copy(
                hrows.at[b], gh_hbm.at[pl.ds(base0, wb)], swh).wait()
        if tail:
            toff = nfull * wb
            tbase = base0 + toff
            cp1 = pltpu.async_copy(
                tdst_hbm.at[dia.at[pl.ds(toff, tail)]],
                drows.at[0].at[pl.ds(0, tail)], sgd)
            cp2 = pltpu.async_copy(
                h_hbm.at[sia.at[pl.ds(toff, tail)]],
                hrows.at[0].at[pl.ds(0, tail)], sgh)
            cp1.wait()
            cp2.wait()
            pltpu.sync_copy(drows.at[0].at[pl.ds(0, tail)],
                            gd_hbm.at[pl.ds(tbase, tail)])
            pltpu.sync_copy(hrows.at[0].at[pl.ds(0, tail)],
                            gh_hbm.at[pl.ds(tbase, tail)])

    return run(tdst, hsrc, dst, src)


def _sc_scatter(m, dst, zeros, n):
    e = dst.shape[0]
    epw = e // _NW
    wb = 128
    nfull = epw // wb
    tail = epw - nfull * wb
    npair = nfull // 2
    odd = nfull % 2
    npt = zeros.shape[0]
    n_pad = npt * 16
    mesh = plsc.VectorSubcoreMesh(core_axis_name="c", subcore_axis_name="s")

    @functools.partial(
        pl.kernel,
        out_type=jax.ShapeDtypeStruct((2, n_pad, 128), jnp.float32),
        mesh=mesh,
        scratch_types=[
            pltpu.VMEM((2, wb, 128), jnp.float32),
            pltpu.VMEM((2, wb), jnp.int32),
            pltpu.VMEM((max(tail, 8), 128), jnp.float32),
            pltpu.VMEM((max(tail, 8),), jnp.int32),
            pltpu.VMEM_SHARED((n_pad, 128), jnp.float32),
            pltpu.SemaphoreType.DMA,
            pltpu.SemaphoreType.DMA,
            pltpu.SemaphoreType.DMA,
        ])
    def run(m_hbm, di_hbm, z_hbm, acc_hbm, mb, dib, mtl, dtl, shared,
            slm, sli, ssc):
        cid = lax.axis_index("c")
        sid = lax.axis_index("s")
        wid = sid * 2 + cid
        base0 = wid * epw

        pltpu.sync_copy(z_hbm, shared.at[pl.ds(sid * npt, npt)])
        plsc.subcore_barrier()

        def one(w, b, drain):
            base = base0 + w * wb
            if drain is not None:
                @pl.when(drain)
                def _():
                    pltpu.make_async_copy(
                        mb.at[b], shared.at[pl.ds(0, wb)], ssc).wait()
            cp1 = pltpu.async_copy(m_hbm.at[pl.ds(base, wb)], mb.at[b], slm)
            cp2 = pltpu.async_copy(di_hbm.at[pl.ds(base, wb)], dib.at[b], sli)
            cp1.wait()
            cp2.wait()
            pltpu.async_copy(mb.at[b], shared.at[dib.at[b]], ssc, add=True)

        def pair(g, carry):
            for b in range(2):
                one(2 * g + b, b, g >= 1)
            return carry

        lax.fori_loop(0, npair, pair, 0)
        if odd:
            one(nfull - 1, 0, npair >= 1)
        for b in range(2):
            pltpu.make_async_copy(
                mb.at[b], shared.at[pl.ds(0, wb)], ssc).wait()
        if tail:
            tbase = base0 + nfull * wb
            pltpu.sync_copy(m_hbm.at[pl.ds(tbase, tail)], mtl)
            pltpu.sync_copy(di_hbm.at[pl.ds(tbase, tail)], dtl)
            pltpu.sync_copy(mtl, shared.at[dtl], add=True)
        plsc.subcore_barrier()
        pltpu.sync_copy(shared.at[pl.ds(sid * npt, npt)],
                        acc_hbm.at[cid].at[pl.ds(sid * npt, npt)])

    return run(m, dst, zeros)


def kernel(h, x, rel_x, r_feat, edge_feat, invar_ligand_shape,
           ligand_shape_emb, topo_out, e_w, params, edge_index):
    n, in_dim = h.shape
    e = rel_x.shape[0]
    pk, pv, pq = params['xk'], params['xv'], params['xq']

    w1k, w1v = pk['W1'], pv['W1']
    wkh, wkt, wki = w1k[32:160], w1k[288:416], w1k[416:448]
    wvh, wvt, wvi = w1v[32:160], w1v[288:416], w1v[416:448]
    wsrc = jnp.concatenate([w1k[160:288], w1v[160:288]], axis=1)
    wef = jnp.concatenate([w1k[0:32], w1v[0:32]], axis=1)
    b1k = pk['b1'][None, :]
    b1v = pv['b1'][None, :]
    s = jnp.kron(jnp.eye(16, dtype=jnp.float32),
                 jnp.ones((8, 1), jnp.float32)) * (1.0 / math.sqrt(8.0))
    z128_64 = jnp.zeros((128, 64), jnp.float32)
    sr = jnp.concatenate([jnp.tile(s, (1, 4)), z128_64], axis=1)
    w2vr = jnp.concatenate([jnp.zeros((128, 16), jnp.float32),
                            jnp.tile(pv['W2'], (1, 3)), z128_64], axis=1)
    b2vr = jnp.concatenate([jnp.ones((16,), jnp.float32),
                            jnp.tile(pv['b2'], 3),
                            jnp.zeros((64,), jnp.float32)])[None, :]
    rmat = jnp.kron(jnp.eye(4, dtype=jnp.float32),
                    jnp.ones((1, 16), jnp.float32))
    rmat = jnp.concatenate([rmat, jnp.zeros((4, 64), jnp.float32)], axis=1)

    src = edge_index[0]
    dst = edge_index[1]

    bn = 1000
    node_ws = [wkh, wkt, wki, b1k, wvh, wvt, wvi, b1v,
               pq['W1'], pq['b1'][None, :], pq['g'][None, :],
               pq['beta'][None, :], pq['W2'], pq['b2'][None, :], wsrc]
    tdst, tsrc = pl.pallas_call(
        _node_body,
        grid=(n // bn,),
        in_specs=[pl.BlockSpec((bn, in_dim), lambda i: (i, 0)),
                  pl.BlockSpec((bn, 128), lambda i: (i, 0)),
                  pl.BlockSpec((bn, 32), lambda i: (i, 0))]
                 + [pl.BlockSpec(w.shape, lambda i, nd=w.ndim: (0,) * nd)
                    for w in node_ws],
        out_specs=[pl.BlockSpec((bn, 256), lambda i: (i, 0)),
                   pl.BlockSpec((bn, 128), lambda i: (i, 0))],
        out_shape=[jax.ShapeDtypeStruct((n, 256), jnp.float32),
                   jax.ShapeDtypeStruct((n, 128), jnp.float32)],
    )(h, topo_out, invar_ligand_shape, *node_ws)

    gdst, gh = _sc_gather(tdst, tsrc, dst, src)

    be = 2000
    edge_ws = [wef,
               pk['g'][None, :], pk['beta'][None, :], pk['W2'], pk['b2'][None, :],
               pv['g'][None, :], pv['beta'][None, :], w2vr, b2vr, sr, rmat]
    m = pl.pallas_call(
        _edge_body,
        grid=(e // be,),
        in_specs=[pl.BlockSpec((be, 256), lambda i: (i, 0)),
                  pl.BlockSpec((be, 128), lambda i: (i, 0)),
                  pl.BlockSpec((be, 16), lambda i: (i, 0)),
                  pl.BlockSpec((be, 16), lambda i: (i, 0)),
                  pl.BlockSpec((be, 1), lambda i: (i, 0)),
                  pl.BlockSpec((be, 3), lambda i: (i, 0))]
                 + [pl.BlockSpec(w.shape, lambda i, nd=w.ndim: (0,) * nd)
                    for w in edge_ws],
        out_specs=pl.BlockSpec((be, 128), lambda i: (i, 0)),
        out_shape=jax.ShapeDtypeStruct((e, 128), jnp.float32),
    )(gdst, gh, edge_feat, r_feat, e_w[:, None], rel_x, *edge_ws)

    npt = ((n + 15) // 16 + 7) // 8 * 8
    zeros = jnp.zeros((npt, 128), jnp.float32)
    acc = _sc_scatter(m, dst, zeros, n)

    bf = 1000
    lse_t = jnp.transpose(ligand_shape_emb, (2, 0, 1))
    wf, wd = params['Wf'], params['Wd']
    head_ws = [wf[0:1], wf[1:17], wf[17:49], wd[0:1], wd[1:17], wd[17:49]]
    out = pl.pallas_call(
        _final_body,
        grid=(n // bf,),
        in_specs=[pl.BlockSpec((2, bf, 128), lambda i: (0, i, 0)),
                  pl.BlockSpec((bf, 3), lambda i: (i, 0)),
                  pl.BlockSpec((3, bf, 32), lambda i: (0, i, 0))]
                 + [pl.BlockSpec(w.shape, lambda i, nd=w.ndim: (0,) * nd)
                    for w in head_ws],
        out_specs=pl.BlockSpec((bf, 3), lambda i: (i, 0)),
        out_shape=jax.ShapeDtypeStruct((n, 3), jnp.float32),
    )(acc, x, lse_t, *head_ws)
    return out


# speedup vs baseline: 34.1673x; 1.0141x over previous
import functools
import math

import jax
import jax.numpy as jnp
from jax import lax
from jax.experimental import pallas as pl
from jax.experimental.pallas import tpu as pltpu
from jax.experimental.pallas import tpu_sc as plsc

_NW = 32
_W = 80
_LN_EPS = 1e-5
_VN_EPS = 1e-6


def _bf16_bits(x):
    r = x.astype(jnp.bfloat16).astype(jnp.float32)
    return lax.bitcast_convert_type(r, jnp.int32)


def _pack2(lo, hi):
    w = lax.shift_right_logical(_bf16_bits(lo), 16) | _bf16_bits(hi)
    return lax.bitcast_convert_type(w, jnp.float32)


def _unpack_lo(w):
    wi = lax.bitcast_convert_type(w, jnp.int32)
    return lax.bitcast_convert_type(lax.shift_left(wi, 16), jnp.float32)


def _unpack_hi(w):
    wi = lax.bitcast_convert_type(w, jnp.int32)
    return lax.bitcast_convert_type(wi & jnp.int32(-65536), jnp.float32)


def _node_body(h_ref, topo_ref, invar_ref,
               wkh_ref, wkt_ref, wki_ref, b1k_ref,
               wvh_ref, wvt_ref, wvi_ref, b1v_ref,
               w1q_ref, b1q_ref, gq_ref, betaq_ref, w2q_ref, b2q_ref,
               wsrc_ref, td_ref, ts_ref):
    h = h_ref[...]
    topo = topo_ref[...]
    invar = invar_ref[...]
    adk = h @ wkh_ref[...] + topo @ wkt_ref[...] + invar @ wki_ref[...] + b1k_ref[...]
    adv = h @ wvh_ref[...] + topo @ wvt_ref[...] + invar @ wvi_ref[...] + b1v_ref[...]
    y = h @ w1q_ref[...] + b1q_ref[...]
    mu = jnp.mean(y, axis=-1, keepdims=True)
    var = jnp.mean((y - mu) ** 2, axis=-1, keepdims=True)
    z = (y - mu) * lax.rsqrt(var + _LN_EPS) * gq_ref[...] + betaq_ref[...]
    z = jnp.maximum(z, 0.0)
    q = z @ w2q_ref[...] + b2q_ref[...]
    td_ref[...] = _pack2(jnp.concatenate([adk, adv], axis=1),
                         jnp.concatenate([q, jnp.zeros_like(q)], axis=1))
    asrc = h @ wsrc_ref[...]
    ts_ref[...] = _pack2(asrc[:, 0:128], asrc[:, 128:256])


def _ln_relu(y, g, beta):
    mu = jnp.mean(y, axis=-1, keepdims=True)
    var = jnp.mean((y - mu) ** 2, axis=-1, keepdims=True)
    z = (y - mu) * lax.rsqrt(var + _LN_EPS) * g + beta
    return jnp.maximum(z, 0.0)


def _edge_body(gd_ref, gh_ref, ef_ref, rf_ref, ew_ref, rx_ref,
               wef_ref,
               gk_ref, betak_ref, w2k_ref, b2k_ref,
               gv_ref, betav_ref, w2vr_ref, b2vr_ref, sr_ref, rmat_ref,
               out_ref):
    wd = gd_ref[...]
    ws = gh_ref[...]
    ad = _unpack_lo(wd)
    qg = _unpack_hi(wd)[:, 0:128]
    wef = wef_ref[...]
    ep = ef_ref[...] @ wef[0:16] + rf_ref[...] @ wef[16:32]
    y1k = ad[:, 0:128] + _unpack_lo(ws) + ep[:, 0:128]
    y1v = ad[:, 128:256] + _unpack_hi(ws) + ep[:, 128:256]
    zk = _ln_relu(y1k, gk_ref[...], betak_ref[...])
    zv = _ln_relu(y1v, gv_ref[...], betav_ref[...])
    kk = zk @ w2k_ref[...] + b2k_ref[...]
    exr = jnp.exp((qg * kk) @ sr_ref[...])
    vsr = zv @ w2vr_ref[...] + b2vr_ref[...]
    rxin = jnp.concatenate(
        [jnp.ones_like(ew_ref[...]), rx_ref[...] * ew_ref[...]], axis=1)
    out_ref[...] = exr * vsr * (rxin @ rmat_ref[...])


def _final_body(acc_ref, x_ref, lse_ref,
                wf0_ref, wfo_ref, wfl_ref, wd0_ref, wdo_ref, wdl_ref,
                out_ref):
    a = acc_ref[0] + acc_ref[1]
    denom = a[:, 0:16] + 1e-16
    ods, ps, ds_ = [], [], []
    for d in range(3):
        od = a[:, 16 * (d + 1):16 * (d + 2)] / denom
        xd = x_ref[:, d:d + 1]
        ld = lse_ref[d]
        ods.append(od)
        ps.append(xd * wf0_ref[...] + od @ wfo_ref[...] + ld @ wfl_ref[...])
        ds_.append(xd * wd0_ref[...] + od @ wdo_ref[...] + ld @ wdl_ref[...])
    dot = ps[0] * ds_[0] + ps[1] * ds_[1] + ps[2] * ds_[2]
    dns = ds_[0] * ds_[0] + ds_[1] * ds_[1] + ds_[2] * ds_[2]
    coef = dot / (dns + _VN_EPS)
    fneg = jnp.where(dot >= 0, 0.0, 0.8)
    cols = []
    for d in range(3):
        vn = ps[d] - fneg * coef * ds_[d]
        cols.append(jnp.mean(ods[d], axis=1, keepdims=True)
                    + jnp.mean(vn, axis=1, keepdims=True))
    out_ref[...] = jnp.concatenate(cols, axis=1)


def _sc_gather(tdst, hsrc, dst, src):
    e = dst.shape[0]
    epw = e // _NW
    wb = 128
    nfull = epw // wb
    tail = epw - nfull * wb
    npair = nfull // 2
    odd = nfull % 2
    dt = tdst.shape[1]
    dh = hsrc.shape[1]
    mesh = plsc.VectorSubcoreMesh(core_axis_name="c", subcore_axis_name="s")

    @functools.partial(
        pl.kernel,
        out_type=[jax.ShapeDtypeStruct((e, dt), jnp.float32),
                  jax.ShapeDtypeStruct((e, dh), jnp.float32)],
        mesh=mesh,
        scratch_types=[
            pltpu.VMEM((epw,), jnp.int32),
            pltpu.VMEM((epw,), jnp.int32),
            pltpu.VMEM((2, wb, dt), jnp.float32),
            pltpu.VMEM((2, wb, dh), jnp.float32),
            pltpu.SemaphoreType.DMA,
            pltpu.SemaphoreType.DMA,
            pltpu.SemaphoreType.DMA,
            pltpu.SemaphoreType.DMA,
        ])
    def run(tdst_hbm, h_hbm, di_hbm, si_hbm, gd_hbm, gh_hbm,
            dia, sia, drows, hrows, sgd, sgh, swd, swh):
        cid = lax.axis_index("c")
        sid = lax.axis_index("s")
        wid = sid * 2 + cid
        base0 = wid * epw
        pltpu.sync_copy(di_hbm.at[pl.ds(base0, epw)], dia)
        pltpu.sync_copy(si_hbm.at[pl.ds(base0, epw)], sia)

        def one(w, b, drain):
            base = base0 + w * wb
            off = w * wb
            if drain is not None:
                @pl.when(drain)
                def _():
                    pltpu.make_async_copy(
                        drows.at[b], gd_hbm.at[pl.ds(base0, wb)], swd).wait()
                    pltpu.make_async_copy(
                        hrows.at[b], gh_hbm.at[pl.ds(base0, wb)], swh).wait()
            cp1 = pltpu.async_copy(
                tdst_hbm.at[dia.at[pl.ds(off, wb)]], drows.at[b], sgd)
            cp2 = pltpu.async_copy(
                h_hbm.at[sia.at[pl.ds(off, wb)]], hrows.at[b], sgh)
            cp1.wait()
            cp2.wait()
            pltpu.async_copy(drows.at[b], gd_hbm.at[pl.ds(base, wb)], swd)
            pltpu.async_copy(hrows.at[b], gh_hbm.at[pl.ds(base, wb)], swh)

        def pair(g, carry):
            for b in range(2):
                one(2 * g + b, b, g >= 1)
            return carry

        lax.fori_loop(0, npair, pair, 0)
        if odd:
            one(nfull - 1, 0, npair >= 1)
        for b in range(2):
            pltpu.make_async_copy(
                drows.at[b], gd_hbm.at[pl.ds(base0, wb)], swd).wait()
            pltpu.make_async_copy(
                hrows.at[b], gh_hbm.at[pl.ds(base0, wb)], swh).wait()
        if tail:
            toff = nfull * wb
            tbase = base0 + toff
            cp1 = pltpu.async_copy(
                tdst_hbm.at[dia.at[pl.ds(toff, tail)]],
                drows.at[0].at[pl.ds(0, tail)], sgd)
            cp2 = pltpu.async_copy(
                h_hbm.at[sia.at[pl.ds(toff, tail)]],
                hrows.at[0].at[pl.ds(0, tail)], sgh)
            cp1.wait()
            cp2.wait()
            pltpu.sync_copy(drows.at[0].at[pl.ds(0, tail)],
                            gd_hbm.at[pl.ds(tbase, tail)])
            pltpu.sync_copy(hrows.at[0].at[pl.ds(0, tail)],
                            gh_hbm.at[pl.ds(tbase, tail)])

    return run(tdst, hsrc, dst, src)


def _sc_scatter(m, dst, zeros, n):
    e = dst.shape[0]
    epw = e // _NW
    wb = 128
    nfull = epw // wb
    tail = epw - nfull * wb
    npair = nfull // 2
    odd = nfull % 2
    npt = zeros.shape[0]
    n_pad = npt * 16
    mesh = plsc.VectorSubcoreMesh(core_axis_name="c", subcore_axis_name="s")

    @functools.partial(
        pl.kernel,
        out_type=jax.ShapeDtypeStruct((2, n_pad, 128), jnp.float32),
        mesh=mesh,
        scratch_types=[
            pltpu.VMEM((2, wb, 128), jnp.float32),
            pltpu.VMEM((2, wb), jnp.int32),
            pltpu.VMEM((max(tail, 8), 128), jnp.float32),
            pltpu.VMEM((max(tail, 8),), jnp.int32),
            pltpu.VMEM_SHARED((n_pad, 128), jnp.float32),
            pltpu.SemaphoreType.DMA,
            pltpu.SemaphoreType.DMA,
            pltpu.SemaphoreType.DMA,
        ])
    def run(m_hbm, di_hbm, z_hbm, acc_hbm, mb, dib, mtl, dtl, shared,
            slm, sli, ssc):
        cid = lax.axis_index("c")
        sid = lax.axis_index("s")
        wid = sid * 2 + cid
        base0 = wid * epw

        pltpu.sync_copy(z_hbm, shared.at[pl.ds(sid * npt, npt)])
        plsc.subcore_barrier()

        def one(w, b, drain):
            base = base0 + w * wb
            if drain is not None:
                @pl.when(drain)
                def _():
                    pltpu.make_async_copy(
                        mb.at[b], shared.at[pl.ds(0, wb)], ssc).wait()
            cp1 = pltpu.async_copy(m_hbm.at[pl.ds(base, wb)], mb.at[b], slm)
            cp2 = pltpu.async_copy(di_hbm.at[pl.ds(base, wb)], dib.at[b], sli)
            cp1.wait()
            cp2.wait()
            pltpu.async_copy(mb.at[b], shared.at[dib.at[b]], ssc, add=True)

        def pair(g, carry):
            for b in range(2):
                one(2 * g + b, b, g >= 1)
            return carry

        lax.fori_loop(0, npair, pair, 0)
        if odd:
            one(nfull - 1, 0, npair >= 1)
        for b in range(2):
            pltpu.make_async_copy(
                mb.at[b], shared.at[pl.ds(0, wb)], ssc).wait()
        if tail:
            tbase = base0 + nfull * wb
            pltpu.sync_copy(m_hbm.at[pl.ds(tbase, tail)], mtl)
            pltpu.sync_copy(di_hbm.at[pl.ds(tbase, tail)], dtl)
            pltpu.sync_copy(mtl, shared.at[dtl], add=True)
        plsc.subcore_barrier()
        pltpu.sync_copy(shared.at[pl.ds(sid * npt, npt)],
                        acc_hbm.at[cid].at[pl.ds(sid * npt, npt)])

    return run(m, dst, zeros)


def kernel(h, x, rel_x, r_feat, edge_feat, invar_ligand_shape,
           ligand_shape_emb, topo_out, e_w, params, edge_index):
    n, in_dim = h.shape
    e = rel_x.shape[0]
    pk, pv, pq = params['xk'], params['xv'], params['xq']

    w1k, w1v = pk['W1'], pv['W1']
    wkh, wkt, wki = w1k[32:160], w1k[288:416], w1k[416:448]
    wvh, wvt, wvi = w1v[32:160], w1v[288:416], w1v[416:448]
    wsrc = jnp.concatenate([w1k[160:288], w1v[160:288]], axis=1)
    wef = jnp.concatenate([w1k[0:32], w1v[0:32]], axis=1)
    b1k = pk['b1'][None, :]
    b1v = pv['b1'][None, :]
    s = jnp.kron(jnp.eye(16, dtype=jnp.float32),
                 jnp.ones((8, 1), jnp.float32)) * (1.0 / math.sqrt(8.0))
    z128_64 = jnp.zeros((128, 64), jnp.float32)
    sr = jnp.concatenate([jnp.tile(s, (1, 4)), z128_64], axis=1)
    w2vr = jnp.concatenate([jnp.zeros((128, 16), jnp.float32),
                            jnp.tile(pv['W2'], (1, 3)), z128_64], axis=1)
    b2vr = jnp.concatenate([jnp.ones((16,), jnp.float32),
                            jnp.tile(pv['b2'], 3),
                            jnp.zeros((64,), jnp.float32)])[None, :]
    rmat = jnp.kron(jnp.eye(4, dtype=jnp.float32),
                    jnp.ones((1, 16), jnp.float32))
    rmat = jnp.concatenate([rmat, jnp.zeros((4, 64), jnp.float32)], axis=1)

    src = edge_index[0]
    dst = edge_index[1]

    bn = 1000
    node_ws = [wkh, wkt, wki, b1k, wvh, wvt, wvi, b1v,
               pq['W1'], pq['b1'][None, :], pq['g'][None, :],
               pq['beta'][None, :], pq['W2'], pq['b2'][None, :], wsrc]
    tdst, tsrc = pl.pallas_call(
        _node_body,
        grid=(n // bn,),
        in_specs=[pl.BlockSpec((bn, in_dim), lambda i: (i, 0)),
                  pl.BlockSpec((bn, 128), lambda i: (i, 0)),
                  pl.BlockSpec((bn, 32), lambda i: (i, 0))]
                 + [pl.BlockSpec(w.shape, lambda i, nd=w.ndim: (0,) * nd)
                    for w in node_ws],
        out_specs=[pl.BlockSpec((bn, 256), lambda i: (i, 0)),
                   pl.BlockSpec((bn, 128), lambda i: (i, 0))],
        out_shape=[jax.ShapeDtypeStruct((n, 256), jnp.float32),
                   jax.ShapeDtypeStruct((n, 128), jnp.float32)],
    )(h, topo_out, invar_ligand_shape, *node_ws)

    gdst, gh = _sc_gather(tdst, tsrc, dst, src)

    be = 3200
    edge_ws = [wef,
               pk['g'][None, :], pk['beta'][None, :], pk['W2'], pk['b2'][None, :],
               pv['g'][None, :], pv['beta'][None, :], w2vr, b2vr, sr, rmat]
    m = pl.pallas_call(
        _edge_body,
        grid=(e // be,),
        in_specs=[pl.BlockSpec((be, 256), lambda i: (i, 0)),
                  pl.BlockSpec((be, 128), lambda i: (i, 0)),
                  pl.BlockSpec((be, 16), lambda i: (i, 0)),
                  pl.BlockSpec((be, 16), lambda i: (i, 0)),
                  pl.BlockSpec((be, 1), lambda i: (i, 0)),
                  pl.BlockSpec((be, 3), lambda i: (i, 0))]
                 + [pl.BlockSpec(w.shape, lambda i, nd=w.ndim: (0,) * nd)
                    for w in edge_ws],
        out_specs=pl.BlockSpec((be, 128), lambda i: (i, 0)),
        out_shape=jax.ShapeDtypeStruct((e, 128), jnp.float32),
    )(gdst, gh, edge_feat, r_feat, e_w[:, None], rel_x, *edge_ws)

    npt = ((n + 15) // 16 + 7) // 8 * 8
    zeros = jnp.zeros((npt, 128), jnp.float32)
    acc = _sc_scatter(m, dst, zeros, n)

    bf = 1000
    lse_t = jnp.transpose(ligand_shape_emb, (2, 0, 1))
    wf, wd = params['Wf'], params['Wd']
    head_ws = [wf[0:1], wf[1:17], wf[17:49], wd[0:1], wd[1:17], wd[17:49]]
    out = pl.pallas_call(
        _final_body,
        grid=(n // bf,),
        in_specs=[pl.BlockSpec((2, bf, 128), lambda i: (0, i, 0)),
                  pl.BlockSpec((bf, 3), lambda i: (i, 0)),
                  pl.BlockSpec((3, bf, 32), lambda i: (0, i, 0))]
                 + [pl.BlockSpec(w.shape, lambda i, nd=w.ndim: (0,) * nd)
                    for w in head_ws],
        out_specs=pl.BlockSpec((bf, 3), lambda i: (i, 0)),
        out_shape=jax.ShapeDtypeStruct((n, 3), jnp.float32),
    )(acc, x, lse_t, *head_ws)
    return out
